# Initial kernel scaffold; baseline (speedup 1.0000x reference)
import jax
import jax.numpy as jnp
from jax.experimental import pallas as pl


def kernel(feat, edge_index, W0, b0, W1, b1, W2, b2, FC1, FC2):
    raise NotImplementedError("write your pallas kernel here")


# speedup vs baseline: 5.8236x; 5.8236x over previous
import functools

import jax
import jax.numpy as jnp
from jax import lax
from jax.experimental import pallas as pl
from jax.experimental.pallas import tpu as pltpu
from jax.experimental.pallas import tpu_sc as plsc

_CH = 128


def _sc_grid(e, nc, ns):
    nw = nc * ns
    q, r = divmod(e, nw * _CH)
    assert r % _CH == 0, (e, nw)
    return nw, q, r // _CH, nw * _CH * q


def _zero_rows(ref, rows, width):
    z16 = jnp.zeros((16,), jnp.float32)

    def body(i, _):
        for j in range(width // 16):
            ref[i, pl.ds(j * 16, 16)] = z16
        return 0

    lax.fori_loop(0, rows, body, 0)


def _row_split(n_nodes, ns):
    rmain = (n_nodes // (ns * 8)) * 8
    tail = n_nodes - ns * rmain
    assert tail % 8 == 0 and tail >= 0
    return rmain, tail


_W = 128
_ZC = 104


def _zero_acc(zbuf_v, acc_sh, s, rmain, tail, ns):
    _zero_rows(zbuf_v, _ZC, _W)
    nz = rmain // _ZC
    assert rmain % _ZC == 0 and tail <= _ZC

    def zb(i, _):
        pltpu.sync_copy(zbuf_v, acc_sh.at[pl.ds(s * rmain + i * _ZC, _ZC)])
        return 0

    lax.fori_loop(0, nz, zb, 0)
    if tail:
        @pl.when(s == ns - 1)
        def _():
            pltpu.sync_copy(zbuf_v.at[pl.ds(0, tail)],
                            acc_sh.at[pl.ds(ns * rmain, tail)])


def _write_out(acc_sh, out_hbm, c, s, rmain, tail, ns):
    pltpu.sync_copy(acc_sh.at[pl.ds(s * rmain, rmain)],
                    out_hbm.at[c, pl.ds(s * rmain, rmain)])
    if tail:
        @pl.when(s == ns - 1)
        def _():
            pltpu.sync_copy(acc_sh.at[pl.ds(ns * rmain, tail)],
                            out_hbm.at[c, pl.ds(ns * rmain, tail)])


def _make_deg(n_nodes, e, nc, ns):
    w = _W
    nw, q, extra, base_extra = _sc_grid(e, nc, ns)
    rmain, tail = _row_split(n_nodes, ns)
    mesh = plsc.VectorSubcoreMesh(core_axis_name="c", subcore_axis_name="s")

    @functools.partial(
        pl.kernel,
        mesh=mesh,
        out_type=jax.ShapeDtypeStruct((nc, n_nodes, w), jnp.float32),
        scratch_types=[
            pltpu.VMEM((_CH,), jnp.int32),
            pltpu.VMEM((_CH, w), jnp.float32),
            pltpu.VMEM((_ZC, w), jnp.float32),
            pltpu.MemorySpace.VMEM_SHARED((n_nodes, w), jnp.float32),
        ],
    )
    def deg_kernel(dst_hbm, out_hbm, dst_v, ones_v, zbuf_v, acc_sh):
        c = lax.axis_index("c")
        s = lax.axis_index("s")
        wid = c * ns + s
        one16 = jnp.ones((16,), jnp.float32)

        def fill_ones(i, _):
            for j in range(w // 16):
                ones_v[i, pl.ds(j * 16, 16)] = one16
            return 0

        lax.fori_loop(0, _CH, fill_ones, 0)
        _zero_acc(zbuf_v, acc_sh, s, rmain, tail, ns)
        plsc.subcore_barrier()

        def chunk(base):
            pltpu.sync_copy(dst_hbm.at[pl.ds(base, _CH)], dst_v)
            pltpu.sync_copy(ones_v, acc_sh.at[dst_v], add=True)

        def body(i, _):
            chunk(wid * (q * _CH) + i * _CH)
            return 0

        lax.fori_loop(0, q, body, 0)
        if extra:
            @pl.when(wid < extra)
            def _():
                chunk(base_extra + wid * _CH)

        plsc.subcore_barrier()
        _write_out(acc_sh, out_hbm, c, s, rmain, tail, ns)

    return deg_kernel


def _make_prop(n_nodes, e, nc, ns):
    w = _W
    nw, q, extra, base_extra = _sc_grid(e, nc, ns)
    rmain, tail = _row_split(n_nodes, ns)
    mesh = plsc.VectorSubcoreMesh(core_axis_name="c", subcore_axis_name="s")

    @functools.partial(
        pl.kernel,
        mesh=mesh,
        out_type=jax.ShapeDtypeStruct((nc, n_nodes, w), jnp.float32),
        scratch_types=[
            pltpu.VMEM((_CH,), jnp.int32),
            pltpu.VMEM((_CH,), jnp.int32),
            pltpu.VMEM((_CH, w), jnp.float32),
            pltpu.VMEM((_ZC, w), jnp.float32),
            pltpu.MemorySpace.VMEM_SHARED((n_nodes, w), jnp.float32),
            pltpu.SemaphoreType.DMA,
        ],
    )
    def prop_kernel(x_hbm, src_hbm, dst_hbm, out_hbm, src_v, dst_v, rows_v,
                    zbuf_v, acc_sh, sem):
        c = lax.axis_index("c")
        s = lax.axis_index("s")
        wid = c * ns + s
        _zero_acc(zbuf_v, acc_sh, s, rmain, tail, ns)
        plsc.subcore_barrier()

        def chunk(base):
            pltpu.sync_copy(src_hbm.at[pl.ds(base, _CH)], src_v)
            pltpu.sync_copy(dst_hbm.at[pl.ds(base, _CH)], dst_v)
            pltpu.async_copy(x_hbm.at[src_v], rows_v, sem).wait()
            pltpu.sync_copy(rows_v, acc_sh.at[dst_v], add=True)

        def body(i, _):
            chunk(wid * (q * _CH) + i * _CH)
            return 0

        lax.fori_loop(0, q, body, 0)
        if extra:
            @pl.when(wid < extra)
            def _():
                chunk(base_extra + wid * _CH)

        plsc.subcore_barrier()
        _write_out(acc_sh, out_hbm, c, s, rmain, tail, ns)

    return prop_kernel


def _matmul_body(hid, ncls, feat_ref, w0_ref, w1_ref, w2_ref, bs_ref, fc1_ref,
                 fc2_ref, degp_ref, y0_ref, z1_ref, z2_ref, norm_ref, bvec_ref):
    f32 = jnp.float32
    fc2 = fc2_ref[...]

    def fold(w_ref, j):
        fj = fc1_ref[pl.ds(j * hid, hid), :]
        return jnp.dot(jnp.dot(w_ref[...], fj, preferred_element_type=f32),
                       fc2, preferred_element_type=f32)

    m0 = fold(w0_ref, 0)
    m1 = fold(w1_ref, 1)
    m2 = fold(w2_ref, 2)
    bs = bs_ref[...]
    bf = (jnp.dot(bs[0:1], fc1_ref[pl.ds(0, hid), :], preferred_element_type=f32)
          + jnp.dot(bs[1:2], fc1_ref[pl.ds(hid, hid), :], preferred_element_type=f32)
          + jnp.dot(bs[2:3], fc1_ref[pl.ds(2 * hid, hid), :], preferred_element_type=f32))
    bvec_ref[...] = jnp.dot(bf, fc2, preferred_element_type=f32)

    dp = degp_ref[...]
    deg = dp[0, :, 0:1] + dp[1, :, 0:1]
    norm = lax.rsqrt(jnp.maximum(deg, 1.0))
    norm_ref[...] = norm
    x = feat_ref[...]
    pad = jnp.zeros((hid, _W - ncls), f32)
    m1p = jnp.concatenate([m1, pad], axis=1)
    m2p = jnp.concatenate([m2, pad], axis=1)
    y0_ref[...] = jnp.dot(x, m0, preferred_element_type=f32)
    z1_ref[...] = norm * jnp.dot(x, m1p, preferred_element_type=f32)
    z2_ref[...] = norm * jnp.dot(x, m2p, preferred_element_type=f32)


def _combine_mid_body(z1_ref, g2_ref, norm_ref, u_ref):
    nrm = norm_ref[...]
    g = g2_ref[...]
    u_ref[...] = z1_ref[...] + (nrm * nrm) * (g[0] + g[1])


def _combine_out_body(ncls, y0_ref, g1_ref, norm_ref, bvec_ref, out_ref):
    g = g1_ref[...]
    gs = (g[0] + g[1])[:, 0:ncls]
    out_ref[...] = y0_ref[...] + norm_ref[...] * gs + bvec_ref[...]


def kernel(feat, edge_index, W0, b0, W1, b1, W2, b2, FC1, FC2):
    n_nodes, d = feat.shape
    e = edge_index.shape[1]
    hid = W0.shape[1]
    ncls = FC2.shape[1]
    f32 = jnp.float32
    info = plsc.get_sparse_core_info()
    nc, ns = info.num_cores, info.num_subcores

    src_arr = edge_index[0]
    dst_arr = edge_index[1]
    degp = _make_deg(n_nodes, e, nc, ns)(dst_arr)

    bs = jnp.stack([b0, b1, b2], axis=0)
    y0, z1p, z2p, norm, bvec = pl.pallas_call(
        functools.partial(_matmul_body, hid, ncls),
        out_shape=[
            jax.ShapeDtypeStruct((n_nodes, ncls), f32),
            jax.ShapeDtypeStruct((n_nodes, _W), f32),
            jax.ShapeDtypeStruct((n_nodes, _W), f32),
            jax.ShapeDtypeStruct((n_nodes, 1), f32),
            jax.ShapeDtypeStruct((1, ncls), f32),
        ],
    )(feat, W0, W1, W2, bs, FC1, FC2, degp)

    prop = _make_prop(n_nodes, e, nc, ns)
    g2 = prop(z2p, src_arr, dst_arr)

    u = pl.pallas_call(
        _combine_mid_body,
        out_shape=jax.ShapeDtypeStruct((n_nodes, _W), f32),
    )(z1p, g2, norm)

    g1 = prop(u, src_arr, dst_arr)

    out = pl.pallas_call(
        functools.partial(_combine_out_body, ncls),
        out_shape=jax.ShapeDtypeStruct((n_nodes, ncls), f32),
    )(y0, g1, norm, bvec)
    return out
